# baseline (speedup 1.0000x reference)
import functools

import jax
import jax.numpy as jnp
from jax.experimental import pallas as pl
from jax.experimental.pallas import tpu as pltpu

_QB = 1024
_KP = 1024
_C = 8


def _tc_body(f_ref, u_ref, tll_ref, tidx_ref, resps_ref, n_ref, elbo_ref,
             cb_ref, *, k_valid, d_feat, n_spikes):
    i = pl.program_id(0)

    @pl.when(i == 0)
    def _init():
        n_ref[...] = jnp.zeros_like(n_ref)
        elbo_ref[0, 0] = jnp.float32(0.0)
        u = u_ref[...]
        u2 = jnp.sum(u * u, axis=1)[None, :]
        kio = jax.lax.broadcasted_iota(jnp.int32, (1, _KP), 1)
        cb = -0.5 * u2 - 0.5 * d_feat * jnp.log(2.0 * jnp.pi)
        cb_ref[...] = jnp.where(kio < k_valid, cb, -jnp.inf)

    f = f_ref[...]
    x2 = jnp.sum(f * f, axis=1, keepdims=True)
    cross = jax.lax.dot_general(
        f, u_ref[...], (((1,), (1,)), ((), ())),
        preferred_element_type=jnp.float32,
    )
    ll = (cross + cb_ref[...]) - 0.5 * x2

    kiota_f = jax.lax.broadcasted_iota(
        jnp.int32, (_QB, _KP), 1).astype(jnp.float32)
    neg_inf = jnp.float32(-jnp.inf)

    work = ll
    m0 = None
    mark = jnp.zeros((_QB, _KP), jnp.float32)
    for c in range(_C):
        m = jnp.max(work, axis=1, keepdims=True)
        cand = jnp.where(work == m, kiota_f, jnp.float32(2 * _KP))
        idx_f = jnp.min(cand, axis=1, keepdims=True)
        onehot = cand == idx_f
        work = jnp.where(onehot, neg_inf, work)
        if c == 0:
            m0 = m
            wc = jnp.ones_like(m)
        else:
            wc = jnp.exp(m - m0)
        mark = jnp.where(onehot, wc, mark)
        tll_ref[:, c:c + 1] = m
        tidx_ref[:, c:c + 1] = idx_f.astype(jnp.int32)

    tll = tll_ref[...]
    w = jnp.exp(tll - m0)
    z = jnp.sum(w, axis=1, keepdims=True)
    inv_z = 1.0 / z
    resps_ref[...] = w * inv_z

    n_ref[...] += jnp.sum(mark * inv_z, axis=0, keepdims=True)

    lse = m0 + jnp.log(z)
    elbo_ref[0, 0] += jnp.sum(lse) / jnp.float32(n_spikes)


def kernel(features, units):
    q, d = features.shape
    k = units.shape[0]
    units_p = jnp.pad(units, ((0, _KP - k), (0, 0)))
    grid = q // _QB

    body = functools.partial(_tc_body, k_valid=k, d_feat=float(d), n_spikes=q)
    tll, tidx, resps, n_vec, elbo = pl.pallas_call(
        body,
        grid=(grid,),
        in_specs=[
            pl.BlockSpec((_QB, d), lambda i: (i, 0)),
            pl.BlockSpec((_KP, d), lambda i: (0, 0)),
        ],
        out_specs=[
            pl.BlockSpec((_QB, _C), lambda i: (i, 0)),
            pl.BlockSpec((_QB, _C), lambda i: (i, 0)),
            pl.BlockSpec((_QB, _C), lambda i: (i, 0)),
            pl.BlockSpec((1, _KP), lambda i: (0, 0)),
            pl.BlockSpec(memory_space=pltpu.SMEM),
        ],
        out_shape=[
            jax.ShapeDtypeStruct((q, _C), jnp.float32),
            jax.ShapeDtypeStruct((q, _C), jnp.int32),
            jax.ShapeDtypeStruct((q, _C), jnp.float32),
            jax.ShapeDtypeStruct((1, _KP), jnp.float32),
            jax.ShapeDtypeStruct((1, 1), jnp.float32),
        ],
        scratch_shapes=[pltpu.VMEM((1, _KP), jnp.float32)],
    )(features, units_p)

    return tll, tidx, resps, n_vec[0, :k], elbo[0, 0]


# speedup vs baseline: 1.0241x; 1.0241x over previous
import functools

import jax
import jax.numpy as jnp
from jax.experimental import pallas as pl
from jax.experimental.pallas import tpu as pltpu

_QB = 512
_KP = 1024
_C = 8


def _tc_body(f_ref, u_ref, tll_ref, tidx_ref, resps_ref, n_ref, elbo_ref,
             cb_ref, *, k_valid, d_feat, n_spikes):
    i = pl.program_id(0)

    @pl.when(i == 0)
    def _init():
        n_ref[...] = jnp.zeros_like(n_ref)
        elbo_ref[0, 0] = jnp.float32(0.0)
        u = u_ref[...]
        u2 = jnp.sum(u * u, axis=1)[None, :]
        kio = jax.lax.broadcasted_iota(jnp.int32, (1, _KP), 1)
        cb = -0.5 * u2 - 0.5 * d_feat * jnp.log(2.0 * jnp.pi)
        cb_ref[...] = jnp.where(kio < k_valid, cb, -jnp.inf)

    f = f_ref[...]
    x2 = jnp.sum(f * f, axis=1, keepdims=True)
    cross = jax.lax.dot_general(
        f, u_ref[...], (((1,), (1,)), ((), ())),
        preferred_element_type=jnp.float32,
    )
    ll = (cross + cb_ref[...]) - 0.5 * x2

    kiota_f = jax.lax.broadcasted_iota(
        jnp.int32, (_QB, _KP), 1).astype(jnp.float32)
    neg_inf = jnp.float32(-jnp.inf)

    work = ll
    m0 = None
    mark = jnp.zeros((_QB, _KP), jnp.float32)
    for c in range(_C):
        m = jnp.max(work, axis=1, keepdims=True)
        cand = jnp.where(work == m, kiota_f, jnp.float32(2 * _KP))
        idx_f = jnp.min(cand, axis=1, keepdims=True)
        onehot = cand == idx_f
        work = jnp.where(onehot, neg_inf, work)
        if c == 0:
            m0 = m
            wc = jnp.ones_like(m)
        else:
            wc = jnp.exp(m - m0)
        mark = jnp.where(onehot, wc, mark)
        tll_ref[:, c:c + 1] = m
        tidx_ref[:, c:c + 1] = idx_f.astype(jnp.int32)

    tll = tll_ref[...]
    w = jnp.exp(tll - m0)
    z = jnp.sum(w, axis=1, keepdims=True)
    inv_z = 1.0 / z
    resps_ref[...] = w * inv_z

    n_ref[...] += jnp.sum(mark * inv_z, axis=0, keepdims=True)

    lse = m0 + jnp.log(z)
    elbo_ref[0, 0] += jnp.sum(lse) / jnp.float32(n_spikes)


def kernel(features, units):
    q, d = features.shape
    k = units.shape[0]
    units_p = jnp.pad(units, ((0, _KP - k), (0, 0)))
    grid = q // _QB

    body = functools.partial(_tc_body, k_valid=k, d_feat=float(d), n_spikes=q)
    tll, tidx, resps, n_vec, elbo = pl.pallas_call(
        body,
        grid=(grid,),
        in_specs=[
            pl.BlockSpec((_QB, d), lambda i: (i, 0)),
            pl.BlockSpec((_KP, d), lambda i: (0, 0)),
        ],
        out_specs=[
            pl.BlockSpec((_QB, _C), lambda i: (i, 0)),
            pl.BlockSpec((_QB, _C), lambda i: (i, 0)),
            pl.BlockSpec((_QB, _C), lambda i: (i, 0)),
            pl.BlockSpec((1, _KP), lambda i: (0, 0)),
            pl.BlockSpec(memory_space=pltpu.SMEM),
        ],
        out_shape=[
            jax.ShapeDtypeStruct((q, _C), jnp.float32),
            jax.ShapeDtypeStruct((q, _C), jnp.int32),
            jax.ShapeDtypeStruct((q, _C), jnp.float32),
            jax.ShapeDtypeStruct((1, _KP), jnp.float32),
            jax.ShapeDtypeStruct((1, 1), jnp.float32),
        ],
        scratch_shapes=[pltpu.VMEM((1, _KP), jnp.float32)],
    )(features, units_p)

    return tll, tidx, resps, n_vec[0, :k], elbo[0, 0]
